# baseline (speedup 1.0000x reference)
import jax
import jax.numpy as jnp
from jax import lax
from jax.experimental import pallas as pl
from jax.experimental.pallas import tpu as pltpu
from jax.experimental.pallas import tpu_sc as plsc

B, N, D = 32, 576, 384
LANES = 16
NC, NS = 2, 16
NW = NC * NS

QUADS = 8
QROWS = N // QUADS
PER_W = B // 4
NB = 3

_sc_mesh = plsc.VectorSubcoreMesh(core_axis_name="c", subcore_axis_name="s")


def _sc_fn(x_hbm, p_hbm, o_hbm):
    def scoped(pos_v, bufs, gsems, ssems):
        cid = lax.axis_index("c")
        sid = lax.axis_index("s")
        wid = sid * NC + cid
        q = wid // 4
        m = wid % 4
        r0 = q * QROWS

        pltpu.sync_copy(p_hbm.at[pl.ds(r0, QROWS), :], pos_v)

        def start_gather(k, j):
            b = m + 4 * k
            pltpu.make_async_copy(
                x_hbm.at[b, pl.ds(r0, QROWS), :], bufs[j], gsems[j]
            ).start()

        def start_scatter(k, j):
            b = m + 4 * k
            pltpu.make_async_copy(
                bufs[j], o_hbm.at[b, pl.ds(r0, QROWS), :], ssems[j]
            ).start()

        def wait_gather(j):
            pltpu.make_async_copy(
                x_hbm.at[0, pl.ds(0, QROWS), :], bufs[j], gsems[j]
            ).wait()

        def wait_scatter(j):
            pltpu.make_async_copy(
                bufs[j], o_hbm.at[0, pl.ds(0, QROWS), :], ssems[j]
            ).wait()

        start_gather(0, 0)
        for k in range(PER_W):
            j = k % NB
            jn = (k + 1) % NB
            if k + 1 < PER_W:
                if k + 1 >= NB:
                    wait_scatter(jn)
                start_gather(k + 1, jn)
            wait_gather(j)

            @plsc.parallel_loop(0, QROWS, step=1, unroll=2)
            def _add(r):
                for c in range(D // LANES):
                    s = pl.ds(c * LANES, LANES)
                    plsc.addupdate(bufs[j].at[r, s], pos_v[r, s])

            start_scatter(k, j)
        for j in range(min(NB, PER_W)):
            wait_scatter(j)

    pl.run_scoped(
        scoped,
        pltpu.VMEM((QROWS, D), jnp.float32),
        [pltpu.VMEM((QROWS, D), jnp.float32) for _ in range(NB)],
        [pltpu.SemaphoreType.DMA for _ in range(NB)],
        [pltpu.SemaphoreType.DMA for _ in range(NB)],
    )


_sc_add = pl.kernel(
    _sc_fn,
    out_type=jax.ShapeDtypeStruct((B, N, D), jnp.float32),
    mesh=_sc_mesh,
    compiler_params=pltpu.CompilerParams(use_tc_tiling_on_sc=True),
)


def kernel(inputs, pos_table):
    return _sc_add(inputs, pos_table)
